# baseline (speedup 1.0000x reference)
import functools

import jax
import jax.numpy as jnp
from jax import lax
from jax.experimental import pallas as pl
from jax.experimental.pallas import tpu as pltpu
from jax.experimental.pallas import tpu_sc as plsc

DIM = 64
PAD = 128
BATCH = 16384
HIST = 200
NC = 2
NS = 16
NW = NC * NS
ROWS_W = BATCH // NW
IB = 8
NIDX = ROWS_W // IB
SPLITS = ((0, 128), (128, HIST - 128))
LANES = 16


def _make_sc_gather():
  mesh = plsc.VectorSubcoreMesh(core_axis_name="c", subcore_axis_name="s")

  @functools.partial(
      pl.kernel,
      mesh=mesh,
      out_type=jax.ShapeDtypeStruct((BATCH, HIST, DIM), jnp.float32),
      compiler_params=pltpu.CompilerParams(use_tc_tiling_on_sc=True),
      scratch_types=[
          pltpu.VMEM((2, IB, HIST), jnp.int32),
          pltpu.VMEM((2, 1, HIST, PAD), jnp.float32),
          pltpu.VMEM((2, 1, HIST, DIM), jnp.float32),
          pltpu.SemaphoreType.DMA,
          pltpu.SemaphoreType.DMA,
          pltpu.SemaphoreType.DMA,
          pltpu.SemaphoreType.DMA,
          pltpu.SemaphoreType.DMA,
          pltpu.SemaphoreType.DMA,
      ],
  )
  def sc_gather(x_hbm, table_hbm, out_hbm, idx_v, pair_v, row_v,
                is0, is1, gs0, gs1, ss0, ss1):
    wid = lax.axis_index("s") * NC + lax.axis_index("c")
    row0 = wid * ROWS_W
    isem = (is0, is1)
    gsem = (gs0, gs1)
    ssem = (ss0, ss1)

    def load_idx(k, a):
      pltpu.async_copy(x_hbm.at[pl.ds(row0 + k * IB, IB)],
                       idx_v.at[a], isem[a])

    def wait_idx(a):
      pltpu.make_async_copy(x_hbm.at[pl.ds(0, IB)], idx_v.at[a],
                            isem[a]).wait()

    def gather(a, u, p):
      handles = [
          pltpu.async_copy(
              table_hbm.at[idx_v.at[a, u, pl.ds(off, n)]],
              pair_v.at[p, 0, pl.ds(off, n)], gsem[p])
          for off, n in SPLITS
      ]
      for h in handles:
        h.wait()

    def compact(p):
      def body(t, carry):
        for j in range(8):
          h = t * 8 + j
          for c in range(DIM // LANES):
            row_v[p, 0, h, pl.ds(c * LANES, LANES)] = (
                pair_v[p, 0, h, pl.ds(c * LANES, LANES)])
        return carry
      lax.fori_loop(0, HIST // 8, body, 0, unroll=False)

    def store(s, p):
      pltpu.async_copy(row_v.at[p],
                       out_hbm.at[pl.ds(row0 + s, 1)], ssem[p])

    def wait_store(p):
      pltpu.make_async_copy(row_v.at[p],
                            out_hbm.at[pl.ds(0, 1)], ssem[p]).wait()

    def run_chunk(k, a):
      wait_idx(a)
      for u in range(IB):
        p = u % 2
        wait_store(p)
        gather(a, u, p)
        compact(p)
        store(k * IB + u, p)
      load_idx(jnp.minimum(k + 2, NIDX - 1), a)

    load_idx(0, 0)
    load_idx(1, 1)
    for p in range(2):
      store(p, p)

    def body(j, carry):
      for a in range(2):
        run_chunk(2 * j + a, a)
      return carry

    lax.fori_loop(0, NIDX // 2, body, 0, unroll=False)

    for p in range(2):
      wait_store(p)
      wait_idx(p)

  return sc_gather


_sc_gather = _make_sc_gather()


@jax.jit
def kernel(x, table):
  tp = jnp.pad(table, ((0, 0), (0, PAD - DIM)))
  return _sc_gather(x.astype(jnp.int32), tp)


# speedup vs baseline: 1.0760x; 1.0760x over previous
import functools

import jax
import jax.numpy as jnp
from jax import lax
from jax.experimental import pallas as pl
from jax.experimental.pallas import tpu as pltpu
from jax.experimental.pallas import tpu_sc as plsc

DIM = 64
PAD = 128
BATCH = 16384
HIST = 200
NC = 2
NS = 16
NW = NC * NS
ROWS_W = BATCH // NW
IB = 8
NIDX = ROWS_W // IB
SPLITS = ((0, 128), (128, HIST - 128))
LANES = 16


def _make_sc_gather():
  mesh = plsc.VectorSubcoreMesh(core_axis_name="c", subcore_axis_name="s")

  @functools.partial(
      pl.kernel,
      mesh=mesh,
      out_type=jax.ShapeDtypeStruct((BATCH, HIST, DIM), jnp.float32),
      compiler_params=pltpu.CompilerParams(use_tc_tiling_on_sc=True),
      scratch_types=[
          pltpu.VMEM((2, IB, HIST), jnp.int32),
          pltpu.VMEM((2, 1, HIST, PAD), jnp.float32),
          pltpu.VMEM((2, 1, HIST, DIM), jnp.float32),
          pltpu.SemaphoreType.DMA,
          pltpu.SemaphoreType.DMA,
          pltpu.SemaphoreType.DMA,
          pltpu.SemaphoreType.DMA,
          pltpu.SemaphoreType.DMA,
          pltpu.SemaphoreType.DMA,
      ],
  )
  def sc_gather(x_hbm, table_hbm, out_hbm, idx_v, pair_v, row_v,
                is0, is1, gs0, gs1, ss0, ss1):
    wid = lax.axis_index("s") * NC + lax.axis_index("c")
    row0 = wid * ROWS_W
    isem = (is0, is1)
    gsem = (gs0, gs1)
    ssem = (ss0, ss1)

    def load_idx(k, a):
      pltpu.async_copy(x_hbm.at[pl.ds(row0 + k * IB, IB)],
                       idx_v.at[a], isem[a])

    def wait_idx(a):
      pltpu.make_async_copy(x_hbm.at[pl.ds(0, IB)], idx_v.at[a],
                            isem[a]).wait()

    def fire_gather(a, u, p):
      for off, n in SPLITS:
        pltpu.async_copy(table_hbm.at[idx_v.at[a, u, pl.ds(off, n)]],
                         pair_v.at[p, 0, pl.ds(off, n)], gsem[p])

    def wait_gather(p):
      for off, n in SPLITS:
        pltpu.make_async_copy(table_hbm.at[idx_v.at[0, 0, pl.ds(off, n)]],
                              pair_v.at[p, 0, pl.ds(off, n)],
                              gsem[p]).wait()

    def compact(p):
      @plsc.parallel_loop(0, HIST, 1, unroll=8)
      def _(h):
        for c in range(DIM // LANES):
          row_v[p, 0, h, pl.ds(c * LANES, LANES)] = (
              pair_v[p, 0, h, pl.ds(c * LANES, LANES)])

    def store(s, p):
      pltpu.async_copy(row_v.at[p],
                       out_hbm.at[pl.ds(row0 + s, 1)], ssem[p])

    def wait_store(p):
      pltpu.make_async_copy(row_v.at[p],
                            out_hbm.at[pl.ds(0, 1)], ssem[p]).wait()

    def run_chunk(k, a):
      for u in range(IB):
        p = u % 2
        wait_gather(p)
        if u < IB - 1:
          fire_gather(a, u + 1, 1 - p)
        else:
          @pl.when(k + 1 < NIDX)
          def _():
            wait_idx(1 - a)
            fire_gather(1 - a, 0, 1 - p)
          load_idx(jnp.minimum(k + 2, NIDX - 1), a)
        wait_store(p)
        compact(p)
        store(k * IB + u, p)

    load_idx(0, 0)
    load_idx(1, 1)
    for p in range(2):
      store(p, p)
    wait_idx(0)
    fire_gather(0, 0, 0)

    def body(j, carry):
      for a in range(2):
        run_chunk(2 * j + a, a)
      return carry

    lax.fori_loop(0, NIDX // 2, body, 0, unroll=False)

    for p in range(2):
      wait_store(p)
      wait_idx(p)

  return sc_gather


_sc_gather = _make_sc_gather()


@jax.jit
def kernel(x, table):
  tp = jnp.pad(table, ((0, 0), (0, PAD - DIM)))
  return _sc_gather(x.astype(jnp.int32), tp)


# speedup vs baseline: 1.2816x; 1.1911x over previous
import functools

import jax
import jax.numpy as jnp
from jax import lax
from jax.experimental import pallas as pl
from jax.experimental.pallas import tpu as pltpu
from jax.experimental.pallas import tpu_sc as plsc

DIM = 64
PAD = 128
BATCH = 16384
HIST = 200
NC = 2
NS = 16
NW = NC * NS
ROWS_W = BATCH // NW
IB = 8
NB = 2
NSUBC = IB // NB
NIDX = ROWS_W // IB
SPLITS = ((0, 128), (128, HIST - 128))


def _make_sc_gather():
  mesh = plsc.VectorSubcoreMesh(core_axis_name="c", subcore_axis_name="s")

  @functools.partial(
      pl.kernel,
      mesh=mesh,
      out_type=jax.ShapeDtypeStruct((BATCH, HIST, PAD), jnp.float32),
      compiler_params=pltpu.CompilerParams(use_tc_tiling_on_sc=True),
      scratch_types=[
          pltpu.VMEM((2, IB, HIST), jnp.int32),
          pltpu.VMEM((2, NB, HIST, PAD), jnp.float32),
          pltpu.SemaphoreType.DMA,
          pltpu.SemaphoreType.DMA,
          pltpu.SemaphoreType.DMA,
          pltpu.SemaphoreType.DMA,
          pltpu.SemaphoreType.DMA,
          pltpu.SemaphoreType.DMA,
      ],
  )
  def sc_gather(x_hbm, table_hbm, out_hbm, idx_v, pair_v,
                is0, is1, gs0, gs1, ss0, ss1):
    wid = lax.axis_index("s") * NC + lax.axis_index("c")
    row0 = wid * ROWS_W
    isem = (is0, is1)
    gsem = (gs0, gs1)
    ssem = (ss0, ss1)

    def load_idx(k, a):
      pltpu.async_copy(x_hbm.at[pl.ds(row0 + k * IB, IB)],
                       idx_v.at[a], isem[a])

    def wait_idx(a):
      pltpu.make_async_copy(x_hbm.at[pl.ds(0, IB)], idx_v.at[a],
                            isem[a]).wait()

    def fire_gather(a, u, p):
      for i in range(NB):
        for off, n in SPLITS:
          pltpu.async_copy(
              table_hbm.at[idx_v.at[a, u * NB + i, pl.ds(off, n)]],
              pair_v.at[p, i, pl.ds(off, n)], gsem[p])

    def wait_gather(p):
      for i in range(NB):
        for off, n in SPLITS:
          pltpu.make_async_copy(
              table_hbm.at[idx_v.at[0, i, pl.ds(off, n)]],
              pair_v.at[p, i, pl.ds(off, n)], gsem[p]).wait()

    def store(s, p):
      pltpu.async_copy(pair_v.at[p],
                       out_hbm.at[pl.ds(row0 + s * NB, NB)], ssem[p])

    def wait_store(p):
      pltpu.make_async_copy(pair_v.at[p],
                            out_hbm.at[pl.ds(0, NB)], ssem[p]).wait()

    def run_chunk(k, a):
      for u in range(NSUBC):
        p = u % 2
        wait_gather(p)
        if u < NSUBC - 1:
          wait_store(1 - p)
          fire_gather(a, u + 1, 1 - p)
        else:
          @pl.when(k + 1 < NIDX)
          def _():
            wait_idx(1 - a)
            wait_store(1 - p)
            fire_gather(1 - a, 0, 1 - p)
          load_idx(jnp.minimum(k + 2, NIDX - 1), a)
        store(k * NSUBC + u, p)

    load_idx(0, 0)
    load_idx(1, 1)
    for p in range(2):
      store(p, p)
    wait_idx(0)
    wait_store(0)
    fire_gather(0, 0, 0)

    def body(j, carry):
      for a in range(2):
        run_chunk(2 * j + a, a)
      return carry

    lax.fori_loop(0, NIDX // 2, body, 0, unroll=False)

    for p in range(2):
      wait_store(p)
      wait_idx(p)

  return sc_gather


_sc_gather = _make_sc_gather()


@jax.jit
def kernel(x, table):
  tp = jnp.pad(table, ((0, 0), (0, PAD - DIM)))
  return _sc_gather(x.astype(jnp.int32), tp)[:, :, :DIM]
